# Initial kernel scaffold; baseline (speedup 1.0000x reference)
import jax
import jax.numpy as jnp
from jax.experimental import pallas as pl


def kernel(sequence, h_hash, s_signs, char_scales, dimension_weights, sketch_bias, char_hash_modifiers, W1, b1, W2, b2):
    raise NotImplementedError("write your pallas kernel here")


# speedup vs baseline: 6212.6373x; 6212.6373x over previous
import jax
import jax.numpy as jnp
from jax import lax
from jax.experimental import pallas as pl

ALPHA = 4
D = 64
SEQ_LEN = 65536
ROWS = 512
COLS = 128


def _count_kernel(seq_ref, idx_ref, sgn_ref, cs_ref, chm_ref, dw_ref, bias_ref,
                  w1t_ref, b1_ref, w2t_ref, b2_ref, out_ref):
    seq = seq_ref[:]

    iu_r = lax.broadcasted_iota(jnp.int32, (COLS, COLS), 0)
    iu_c = lax.broadcasted_iota(jnp.int32, (COLS, COLS), 1)
    U = (iu_r < iu_c).astype(jnp.float32)
    il_r = lax.broadcasted_iota(jnp.int32, (ROWS, ROWS), 0)
    il_c = lax.broadcasted_iota(jnp.int32, (ROWS, ROWS), 1)
    L = (il_c < il_r).astype(jnp.float32)

    O = [(seq == a).astype(jnp.float32) for a in range(ALPHA)]

    RS1 = jnp.concatenate(
        [jnp.sum(O[a], axis=1, keepdims=True) for a in range(ALPHA)], axis=1)
    PRE1 = jnp.dot(L, RS1, preferred_element_type=jnp.float32)

    P = [jnp.dot(O[a], U, preferred_element_type=jnp.float32) + PRE1[:, a:a + 1]
         for a in range(ALPHA)]

    G = [[P[a] * O[b] for b in range(ALPHA)] for a in range(ALPHA)]
    RS2 = jnp.concatenate(
        [jnp.sum(G[a][b], axis=1, keepdims=True)
         for a in range(ALPHA) for b in range(ALPHA)], axis=1)
    PRE2 = jnp.dot(L, RS2, preferred_element_type=jnp.float32)

    cols = []
    for a in range(ALPHA):
        for b in range(ALPHA):
            k = a * ALPHA + b
            Qab = jnp.dot(G[a][b], U, preferred_element_type=jnp.float32) + PRE2[:, k:k + 1]
            for c in range(ALPHA):
                cols.append(jnp.sum(Qab * O[c], axis=1, keepdims=True))
    R3 = jnp.concatenate(cols, axis=1)
    c3row = jnp.sum(R3, axis=0, keepdims=True)

    c1row = jnp.sum(RS1, axis=0, keepdims=True)

    ed = lax.broadcasted_iota(jnp.int32, (D, D), 1)
    E = jnp.where(ed == idx_ref[:], sgn_ref[:], 0.0)
    baseline = jnp.dot(c3row, E, preferred_element_type=jnp.float32)

    inv_n = 1.0 / SEQ_LEN
    scaling = jnp.sum(c1row * cs_ref[:], axis=1, keepdims=True) * inv_n
    mods = jnp.dot(c1row, chm_ref[:], preferred_element_type=jnp.float32) * inv_n

    enhanced = (baseline * dw_ref[:] + bias_ref[:]) * scaling + mods
    hidden = jnp.maximum(
        jnp.dot(enhanced, w1t_ref[:], preferred_element_type=jnp.float32) + b1_ref[:], 0.0)
    out = jnp.dot(hidden, w2t_ref[:], preferred_element_type=jnp.float32) + b2_ref[:]
    out_ref[:] = out


def kernel(sequence, h_hash, s_signs, char_scales, dimension_weights, sketch_bias,
           char_hash_modifiers, W1, b1, W2, b2):
    seq2d = sequence.reshape(ROWS, COLS)

    idx64 = jnp.reshape(
        (h_hash[0][:, None, None] + h_hash[1][None, :, None] + h_hash[2][None, None, :]) % D,
        (D, 1)).astype(jnp.int32)
    sgn64 = jnp.reshape(
        s_signs[0][:, None, None] * s_signs[1][None, :, None] * s_signs[2][None, None, :],
        (D, 1))

    out = pl.pallas_call(
        _count_kernel,
        out_shape=jax.ShapeDtypeStruct((1, D), jnp.float32),
    )(seq2d, idx64, sgn64,
      char_scales.reshape(1, ALPHA), char_hash_modifiers,
      dimension_weights.reshape(1, D), sketch_bias.reshape(1, D),
      W1.T, b1.reshape(1, D), W2.T, b2.reshape(1, D))
    return out.reshape(D)
